# Initial kernel scaffold; baseline (speedup 1.0000x reference)
import jax
import jax.numpy as jnp
from jax.experimental import pallas as pl


def kernel(idx, emb_weight):
    raise NotImplementedError("write your pallas kernel here")


# speedup vs baseline: 4.8083x; 4.8083x over previous
import functools

import jax
import jax.numpy as jnp
from jax import lax
from jax.experimental import pallas as pl
from jax.experimental.pallas import tpu as pltpu
from jax.experimental.pallas import tpu_sc as plsc

DIM = 32
B_TOTAL = 16384 * 200
NC, NS = 2, 16
NW = NC * NS
BPW = B_TOTAL // NW
IDX_W = 128
G = 8
ROWS_PER_IT = G * IDX_W
NIT = BPW // ROWS_PER_IT

_mesh = plsc.VectorSubcoreMesh(core_axis_name="c", subcore_axis_name="s")


@functools.partial(
    pl.kernel,
    mesh=_mesh,
    out_type=jax.ShapeDtypeStruct((B_TOTAL, DIM), jnp.float32),
    scratch_types=[
        pltpu.VMEM((G, IDX_W), jnp.int32),
        pltpu.VMEM((ROWS_PER_IT, DIM), jnp.float32),
        pltpu.SemaphoreType.DMA,
    ],
    compiler_params=pltpu.CompilerParams(use_tc_tiling_on_sc=False),
)
def _gather_kernel(table_hbm, idx_hbm, out_hbm, idx_v, rows_v, sem):
    wid = lax.axis_index("s") * NC + lax.axis_index("c")
    base_irow = wid * (BPW // IDX_W)

    def body(it, carry):
        irow = base_irow + it * G
        pltpu.sync_copy(idx_hbm.at[pl.ds(irow, G)], idx_v)
        copies = [
            pltpu.async_copy(
                table_hbm.at[idx_v.at[j]],
                rows_v.at[pl.ds(j * IDX_W, IDX_W)],
                sem,
            )
            for j in range(G)
        ]
        for c in copies:
            c.wait()
        pltpu.sync_copy(rows_v, out_hbm.at[pl.ds(irow * IDX_W, ROWS_PER_IT)])
        return carry

    lax.fori_loop(0, NIT, body, 0)


def kernel(idx, emb_weight):
    idx32 = idx.reshape(B_TOTAL // IDX_W, IDX_W).astype(jnp.int32)
    out = _gather_kernel(emb_weight, idx32)
    return out.reshape(idx.shape + (DIM,))


# speedup vs baseline: 5.0471x; 1.0497x over previous
import functools

import jax
import jax.numpy as jnp
from jax import lax
from jax.experimental import pallas as pl
from jax.experimental.pallas import tpu as pltpu
from jax.experimental.pallas import tpu_sc as plsc

DIM = 32
B_TOTAL = 16384 * 200
NC, NS = 2, 16
NW = NC * NS
BPW = B_TOTAL // NW
IDX_W = 128
G = 8
ROWS_PER_IT = G * IDX_W
NIT = BPW // ROWS_PER_IT

_mesh = plsc.VectorSubcoreMesh(core_axis_name="c", subcore_axis_name="s")


@functools.partial(
    pl.kernel,
    mesh=_mesh,
    out_type=jax.ShapeDtypeStruct((B_TOTAL, DIM), jnp.float32),
    scratch_types=[
        pltpu.VMEM((2, G, IDX_W), jnp.int32),
        pltpu.VMEM((2, ROWS_PER_IT, DIM), jnp.float32),
        pltpu.SemaphoreType.DMA,
        pltpu.SemaphoreType.DMA,
        pltpu.SemaphoreType.DMA,
    ],
    compiler_params=pltpu.CompilerParams(use_tc_tiling_on_sc=False),
)
def _gather_kernel(table_hbm, idx_hbm, out_hbm, idx_v, rows_v, gsem0, gsem1,
                   ssem):
    wid = lax.axis_index("s") * NC + lax.axis_index("c")
    base_irow = wid * (BPW // IDX_W)
    gsem = (gsem0, gsem1)

    def load_idx(it, b):
        pltpu.sync_copy(idx_hbm.at[pl.ds(base_irow + it * G, G)], idx_v.at[b])

    def start_gather(b):
        for j in range(G):
            pltpu.async_copy(
                table_hbm.at[idx_v.at[b, j]],
                rows_v.at[b, pl.ds(j * IDX_W, IDX_W)],
                gsem[b],
            )

    def wait_gather(b):
        pltpu.make_async_copy(
            out_hbm.at[pl.ds(0, ROWS_PER_IT)],
            rows_v.at[b],
            gsem[b],
        ).wait()

    def start_store(it, b):
        pltpu.async_copy(
            rows_v.at[b],
            out_hbm.at[pl.ds((base_irow + it * G) * IDX_W, ROWS_PER_IT)],
            ssem,
        )

    def wait_store(b):
        pltpu.make_async_copy(
            out_hbm.at[pl.ds(0, ROWS_PER_IT)],
            rows_v.at[b],
            ssem,
        ).wait()

    load_idx(0, 0)
    start_gather(0)
    load_idx(1, 1)
    start_gather(1)
    wait_gather(0)
    start_store(0, 0)

    def pair_body(i2, carry):
        for b, dit in ((1, 1), (0, 2)):
            it = 2 * i2 + dit
            load_idx(it + 1, 1 - b)
            wait_store(1 - b)
            start_gather(1 - b)
            wait_gather(b)
            start_store(it, b)
        return carry

    lax.fori_loop(0, (NIT - 2) // 2, pair_body, 0)

    wait_store(0)
    wait_gather(1)
    start_store(NIT - 1, 1)
    wait_store(1)


def kernel(idx, emb_weight):
    idx32 = idx.reshape(B_TOTAL // IDX_W, IDX_W).astype(jnp.int32)
    out = _gather_kernel(emb_weight, idx32)
    return out.reshape(idx.shape + (DIM,))
